# baseline (speedup 1.0000x reference)
import functools

import jax
import jax.numpy as jnp
from jax import lax
from jax.experimental import pallas as pl
from jax.experimental.pallas import tpu as pltpu
from jax.experimental.pallas import tpu_sc as plsc


@functools.lru_cache(maxsize=None)
def _make_deg(E, N, NC, NS):
  NW = NC * NS
  per = E // NW
  mesh = plsc.VectorSubcoreMesh(core_axis_name="c", subcore_axis_name="s")

  @functools.partial(
      pl.kernel,
      mesh=mesh,
      compiler_params=pltpu.CompilerParams(needs_layout_passes=False),
      out_type=[
          jax.ShapeDtypeStruct((NW, N), jnp.float32),
          jax.ShapeDtypeStruct((E,), jnp.int32),
      ],
      scratch_types=[
          pltpu.VMEM((per,), jnp.int32),
          pltpu.VMEM((per,), jnp.int32),
          pltpu.VMEM((per,), jnp.int32),
          pltpu.VMEM((N,), jnp.float32),
      ],
  )
  def deg_kernel(ei_hbm, out_hbm, pk_hbm, schunk, dchunk, pchunk, cnt):
    wid = lax.axis_index("s") * NC + lax.axis_index("c")
    pltpu.sync_copy(ei_hbm.at[pl.ds(wid * per, per)], schunk)
    pltpu.sync_copy(ei_hbm.at[pl.ds(E + wid * per, per)], dchunk)

    zero = jnp.zeros((16,), jnp.float32)

    @plsc.parallel_loop(0, N // 16, unroll=8)
    def zbody(i):
      cnt[pl.ds(i * 16, 16)] = zero

    ones = jnp.ones((16,), jnp.float32)

    @plsc.parallel_loop(0, per // 16, unroll=8)
    def body(i):
      s = schunk[pl.ds(i * 16, 16)]
      d = dchunk[pl.ds(i * 16, 16)]
      pchunk[pl.ds(i * 16, 16)] = (s << 14) | d
      plsc.addupdate_scatter(cnt, [d], ones)
    pltpu.sync_copy(cnt, out_hbm.at[wid])
    pltpu.sync_copy(pchunk, pk_hbm.at[pl.ds(wid * per, per)])

  return deg_kernel


@functools.lru_cache(maxsize=None)
def _make_scatter(E, N, H, NC, NS, C):
  NW = NC * NS
  R = H // NW
  P = R // 2
  mesh = plsc.VectorSubcoreMesh(core_axis_name="c", subcore_axis_name="s")

  @functools.partial(
      pl.kernel,
      mesh=mesh,
      compiler_params=pltpu.CompilerParams(needs_layout_passes=False),
      out_type=jax.ShapeDtypeStruct((H * N,), jnp.float32),
      scratch_types=[
          pltpu.VMEM((P * N,), jnp.int32),
          pltpu.VMEM((R * N,), jnp.float32),
          pltpu.VMEM((C,), jnp.int32),
          pltpu.VMEM((C,), jnp.int32),
          pltpu.SemaphoreType.DMA,
          pltpu.SemaphoreType.DMA,
      ],
  )
  def scat_kernel(hp_hbm, pk_hbm, out_hbm,
                  hrows, acc, ech0, ech1, sem0, sem1):
    wid = lax.axis_index("s") * NC + lax.axis_index("c")
    pltpu.async_copy(pk_hbm.at[pl.ds(0, C)], ech0, sem0)
    pltpu.sync_copy(hp_hbm.at[pl.ds(wid * (P * N), P * N)], hrows)

    zero = jnp.zeros((16,), jnp.float32)

    @plsc.parallel_loop(0, (R * N) // 16, unroll=8)
    def zbody(i):
      acc[pl.ds(i * 16, 16)] = zero

    nchunk = E // C

    def start(ci, eref, sem):
      pltpu.async_copy(pk_hbm.at[pl.ds(ci * C, C)], eref, sem)

    def waitbuf(eref, sem):
      pltpu.make_async_copy(pk_hbm.at[pl.ds(0, C)], eref, sem).wait()

    himask = jnp.full((16,), -65536, jnp.int32)
    dmask = jnp.full((16,), 16383, jnp.int32)

    def compute(eref):
      @plsc.parallel_loop(0, C // 16, unroll=8)
      def vb(i):
        e = eref[pl.ds(i * 16, 16)]
        s = e >> 14
        d = e & dmask
        for p in range(P):
          v = plsc.load_gather(hrows, [s + (p * N)])
          hi = plsc.bitcast(v & himask, jnp.float32)
          lo = plsc.bitcast(v << 16, jnp.float32)
          plsc.addupdate_scatter(acc, [d + (p * N)], hi)
          plsc.addupdate_scatter(acc, [d + ((P + p) * N)], lo)

    def group(gi, carry):
      c0 = 2 * gi
      start(c0 + 1, ech1, sem1)
      waitbuf(ech0, sem0)
      compute(ech0)
      start(lax.rem(c0 + 2, nchunk), ech0, sem0)
      waitbuf(ech1, sem1)
      compute(ech1)
      return carry

    lax.fori_loop(0, nchunk // 2, group, 0)
    waitbuf(ech0, sem0)
    pltpu.sync_copy(acc.at[pl.ds(0, P * N)],
                    out_hbm.at[pl.ds(wid * (P * N), P * N)])
    pltpu.sync_copy(acc.at[pl.ds(P * N, P * N)],
                    out_hbm.at[pl.ds((NW + wid) * (P * N), P * N)])

  return scat_kernel


def _ka_body(x_ref, w1_ref, b1_ref, cnt_ref, dis_ref, hp_ref):
  z = lax.dot_general(w1_ref[...], x_ref[...], (((1,), (1,)), ((), ())),
                      preferred_element_type=jnp.float32)
  deg = jnp.sum(cnt_ref[...], axis=0, keepdims=True) + 1.0
  dis = lax.rsqrt(deg)
  dis_ref[...] = dis
  hts = jnp.sin(z + b1_ref[...]) * dis
  hh = hts.shape[0] // 2
  top = lax.bitcast_convert_type(
      hts[:hh].astype(jnp.bfloat16), jnp.uint16).astype(jnp.uint32)
  bot = lax.bitcast_convert_type(
      hts[hh:].astype(jnp.bfloat16), jnp.uint16).astype(jnp.uint32)
  hp_ref[...] = lax.bitcast_convert_type((top << 16) | bot, jnp.int32)


def _kb_body(accT_ref, hp_ref, dis_ref, wg_ref, bg_ref, wc_ref, bc_ref, y_ref):
  dis = dis_ref[...]
  hh = hp_ref.shape[0]
  hp = lax.bitcast_convert_type(hp_ref[...], jnp.uint32)
  hi = lax.bitcast_convert_type(hp & jnp.uint32(0xFFFF0000), jnp.float32)
  lo = lax.bitcast_convert_type(hp << 16, jnp.float32)
  m_top = (accT_ref[:hh, :] + hi) * dis
  m_bot = (accT_ref[hh:, :] + lo) * dis
  g = (lax.dot_general(wg_ref[:, :hh], m_top, (((1,), (0,)), ((), ())),
                       preferred_element_type=jnp.float32)
       + lax.dot_general(wg_ref[:, hh:], m_bot, (((1,), (0,)), ((), ())),
                         preferred_element_type=jnp.float32))
  g = jnp.maximum(g + bg_ref[...], 0.0)
  y = lax.dot_general(wc_ref[...], g, (((1,), (0,)), ((), ())),
                      preferred_element_type=jnp.float32)
  y_ref[...] = y + bc_ref[...]


def kernel(x, edge_index, W1, b1, Wg, bg, Wc, bc):
  N, D = x.shape
  H = W1.shape[0]
  OUT = Wc.shape[0]
  E = edge_index.shape[1]
  NC, NS = 2, 16
  NW = NC * NS

  cnt, pk = _make_deg(E, N, NC, NS)(edge_index.reshape(2 * E))

  BN = 512
  grid_a = (pl.cdiv(N, BN),)
  dis2d, hp = pl.pallas_call(
      _ka_body,
      grid=grid_a,
      in_specs=[
          pl.BlockSpec((BN, D), lambda j: (j, 0)),
          pl.BlockSpec((H, D), lambda j: (0, 0)),
          pl.BlockSpec((H, 1), lambda j: (0, 0)),
          pl.BlockSpec((NW, BN), lambda j: (0, j)),
      ],
      out_specs=[
          pl.BlockSpec((1, BN), lambda j: (0, j)),
          pl.BlockSpec((H // 2, BN), lambda j: (0, j)),
      ],
      out_shape=[
          jax.ShapeDtypeStruct((1, N), jnp.float32),
          jax.ShapeDtypeStruct((H // 2, N), jnp.int32),
      ],
  )(x, W1, b1.reshape(H, 1), cnt)

  C = 16000
  accT_flat = _make_scatter(E, N, H, NC, NS, C)(
      hp.reshape((H // 2) * N), pk)
  accT = accT_flat.reshape(H, N)

  grid_b = (pl.cdiv(N, BN),)
  y2d = pl.pallas_call(
      _kb_body,
      grid=grid_b,
      in_specs=[
          pl.BlockSpec((H, BN), lambda j: (0, j)),
          pl.BlockSpec((H // 2, BN), lambda j: (0, j)),
          pl.BlockSpec((1, BN), lambda j: (0, j)),
          pl.BlockSpec((H, H), lambda j: (0, 0)),
          pl.BlockSpec((H, 1), lambda j: (0, 0)),
          pl.BlockSpec((OUT, H), lambda j: (0, 0)),
          pl.BlockSpec((OUT, 1), lambda j: (0, 0)),
      ],
      out_specs=pl.BlockSpec((OUT, BN), lambda j: (0, j)),
      out_shape=jax.ShapeDtypeStruct((OUT, N), jnp.float32),
  )(accT, hp, dis2d, Wg, bg.reshape(H, 1), Wc, bc.reshape(OUT, 1))

  return jnp.squeeze(y2d.T)


# speedup vs baseline: 1.0120x; 1.0120x over previous
import functools

import jax
import jax.numpy as jnp
from jax import lax
from jax.experimental import pallas as pl
from jax.experimental.pallas import tpu as pltpu
from jax.experimental.pallas import tpu_sc as plsc


@functools.lru_cache(maxsize=None)
def _make_deg(E, N, NC, NS):
  NW = NC * NS
  per = E // NW
  mesh = plsc.VectorSubcoreMesh(core_axis_name="c", subcore_axis_name="s")

  @functools.partial(
      pl.kernel,
      mesh=mesh,
      compiler_params=pltpu.CompilerParams(needs_layout_passes=False),
      out_type=[
          jax.ShapeDtypeStruct((NW, N), jnp.float32),
          jax.ShapeDtypeStruct((E,), jnp.int32),
      ],
      scratch_types=[
          pltpu.VMEM((per,), jnp.int32),
          pltpu.VMEM((per,), jnp.int32),
          pltpu.VMEM((per,), jnp.int32),
          pltpu.VMEM((N,), jnp.float32),
      ],
  )
  def deg_kernel(ei_hbm, out_hbm, pk_hbm, schunk, dchunk, pchunk, cnt):
    wid = lax.axis_index("s") * NC + lax.axis_index("c")
    pltpu.sync_copy(ei_hbm.at[pl.ds(wid * per, per)], schunk)
    pltpu.sync_copy(ei_hbm.at[pl.ds(E + wid * per, per)], dchunk)

    zero = jnp.zeros((16,), jnp.float32)

    @plsc.parallel_loop(0, N // 16, unroll=8)
    def zbody(i):
      cnt[pl.ds(i * 16, 16)] = zero

    ones = jnp.ones((16,), jnp.float32)

    @plsc.parallel_loop(0, per // 16, unroll=8)
    def body(i):
      s = schunk[pl.ds(i * 16, 16)]
      d = dchunk[pl.ds(i * 16, 16)]
      pchunk[pl.ds(i * 16, 16)] = (s << 14) | d
      plsc.addupdate_scatter(cnt, [d], ones)
    pltpu.sync_copy(cnt, out_hbm.at[wid])
    pltpu.sync_copy(pchunk, pk_hbm.at[pl.ds(wid * per, per)])

  return deg_kernel


@functools.lru_cache(maxsize=None)
def _make_scatter(E, N, H, NC, NS, C):
  NW = NC * NS
  R = H // NW
  P = R // 2
  mesh = plsc.VectorSubcoreMesh(core_axis_name="c", subcore_axis_name="s")

  @functools.partial(
      pl.kernel,
      mesh=mesh,
      compiler_params=pltpu.CompilerParams(needs_layout_passes=False),
      out_type=jax.ShapeDtypeStruct((H * N,), jnp.float32),
      scratch_types=[
          pltpu.VMEM((P * N,), jnp.int32),
          pltpu.VMEM((R * N,), jnp.float32),
          pltpu.VMEM((C,), jnp.int32),
          pltpu.VMEM((C,), jnp.int32),
          pltpu.SemaphoreType.DMA,
          pltpu.SemaphoreType.DMA,
      ],
  )
  def scat_kernel(hp_hbm, pk_hbm, out_hbm,
                  hrows, acc, ech0, ech1, sem0, sem1):
    wid = lax.axis_index("s") * NC + lax.axis_index("c")
    pltpu.async_copy(pk_hbm.at[pl.ds(0, C)], ech0, sem0)
    pltpu.sync_copy(hp_hbm.at[pl.ds(wid * (P * N), P * N)], hrows)

    zero = jnp.zeros((16,), jnp.float32)

    @plsc.parallel_loop(0, (R * N) // 16, unroll=8)
    def zbody(i):
      acc[pl.ds(i * 16, 16)] = zero

    nchunk = E // C

    def start(ci, eref, sem):
      pltpu.async_copy(pk_hbm.at[pl.ds(ci * C, C)], eref, sem)

    def waitbuf(eref, sem):
      pltpu.make_async_copy(pk_hbm.at[pl.ds(0, C)], eref, sem).wait()

    himask = jnp.full((16,), -65536, jnp.int32)
    dmask = jnp.full((16,), 16383, jnp.int32)

    def compute(eref):
      @plsc.parallel_loop(0, C // 16, unroll=4)
      def vb(i):
        e = eref[pl.ds(i * 16, 16)]
        s = e >> 14
        d = e & dmask
        for p in range(P):
          v = plsc.load_gather(hrows, [s + (p * N)])
          hi = plsc.bitcast(v & himask, jnp.float32)
          lo = plsc.bitcast(v << 16, jnp.float32)
          plsc.addupdate_scatter(acc, [d + (p * N)], hi)
          plsc.addupdate_scatter(acc, [d + ((P + p) * N)], lo)

    def group(gi, carry):
      c0 = 2 * gi
      start(c0 + 1, ech1, sem1)
      waitbuf(ech0, sem0)
      compute(ech0)
      start(lax.rem(c0 + 2, nchunk), ech0, sem0)
      waitbuf(ech1, sem1)
      compute(ech1)
      return carry

    lax.fori_loop(0, nchunk // 2, group, 0)
    waitbuf(ech0, sem0)
    pltpu.sync_copy(acc.at[pl.ds(0, P * N)],
                    out_hbm.at[pl.ds(wid * (P * N), P * N)])
    pltpu.sync_copy(acc.at[pl.ds(P * N, P * N)],
                    out_hbm.at[pl.ds((NW + wid) * (P * N), P * N)])

  return scat_kernel


def _ka_body(x_ref, w1_ref, b1_ref, cnt_ref, dis_ref, hp_ref):
  z = lax.dot_general(w1_ref[...], x_ref[...], (((1,), (1,)), ((), ())),
                      preferred_element_type=jnp.float32)
  deg = jnp.sum(cnt_ref[...], axis=0, keepdims=True) + 1.0
  dis = lax.rsqrt(deg)
  dis_ref[...] = dis
  hts = jnp.sin(z + b1_ref[...]) * dis
  hh = hts.shape[0] // 2
  top = lax.bitcast_convert_type(
      hts[:hh].astype(jnp.bfloat16), jnp.uint16).astype(jnp.uint32)
  bot = lax.bitcast_convert_type(
      hts[hh:].astype(jnp.bfloat16), jnp.uint16).astype(jnp.uint32)
  hp_ref[...] = lax.bitcast_convert_type((top << 16) | bot, jnp.int32)


def _kb_body(accT_ref, hp_ref, dis_ref, wg_ref, bg_ref, wc_ref, bc_ref, y_ref):
  dis = dis_ref[...]
  hh = hp_ref.shape[0]
  hp = lax.bitcast_convert_type(hp_ref[...], jnp.uint32)
  hi = lax.bitcast_convert_type(hp & jnp.uint32(0xFFFF0000), jnp.float32)
  lo = lax.bitcast_convert_type(hp << 16, jnp.float32)
  m_top = (accT_ref[:hh, :] + hi) * dis
  m_bot = (accT_ref[hh:, :] + lo) * dis
  g = (lax.dot_general(wg_ref[:, :hh], m_top, (((1,), (0,)), ((), ())),
                       preferred_element_type=jnp.float32)
       + lax.dot_general(wg_ref[:, hh:], m_bot, (((1,), (0,)), ((), ())),
                         preferred_element_type=jnp.float32))
  g = jnp.maximum(g + bg_ref[...], 0.0)
  y = lax.dot_general(wc_ref[...], g, (((1,), (0,)), ((), ())),
                      preferred_element_type=jnp.float32)
  y_ref[...] = y + bc_ref[...]


def kernel(x, edge_index, W1, b1, Wg, bg, Wc, bc):
  N, D = x.shape
  H = W1.shape[0]
  OUT = Wc.shape[0]
  E = edge_index.shape[1]
  NC, NS = 2, 16
  NW = NC * NS

  cnt, pk = _make_deg(E, N, NC, NS)(edge_index.reshape(2 * E))

  BN = 512
  grid_a = (pl.cdiv(N, BN),)
  dis2d, hp = pl.pallas_call(
      _ka_body,
      grid=grid_a,
      in_specs=[
          pl.BlockSpec((BN, D), lambda j: (j, 0)),
          pl.BlockSpec((H, D), lambda j: (0, 0)),
          pl.BlockSpec((H, 1), lambda j: (0, 0)),
          pl.BlockSpec((NW, BN), lambda j: (0, j)),
      ],
      out_specs=[
          pl.BlockSpec((1, BN), lambda j: (0, j)),
          pl.BlockSpec((H // 2, BN), lambda j: (0, j)),
      ],
      out_shape=[
          jax.ShapeDtypeStruct((1, N), jnp.float32),
          jax.ShapeDtypeStruct((H // 2, N), jnp.int32),
      ],
  )(x, W1, b1.reshape(H, 1), cnt)

  C = 16000
  accT_flat = _make_scatter(E, N, H, NC, NS, C)(
      hp.reshape((H // 2) * N), pk)
  accT = accT_flat.reshape(H, N)

  grid_b = (pl.cdiv(N, BN),)
  y2d = pl.pallas_call(
      _kb_body,
      grid=grid_b,
      in_specs=[
          pl.BlockSpec((H, BN), lambda j: (0, j)),
          pl.BlockSpec((H // 2, BN), lambda j: (0, j)),
          pl.BlockSpec((1, BN), lambda j: (0, j)),
          pl.BlockSpec((H, H), lambda j: (0, 0)),
          pl.BlockSpec((H, 1), lambda j: (0, 0)),
          pl.BlockSpec((OUT, H), lambda j: (0, 0)),
          pl.BlockSpec((OUT, 1), lambda j: (0, 0)),
      ],
      out_specs=pl.BlockSpec((OUT, BN), lambda j: (0, j)),
      out_shape=jax.ShapeDtypeStruct((OUT, N), jnp.float32),
  )(accT, hp, dis2d, Wg, bg.reshape(H, 1), Wc, bc.reshape(OUT, 1))

  return jnp.squeeze(y2d.T)


# speedup vs baseline: 1.0193x; 1.0073x over previous
import functools

import jax
import jax.numpy as jnp
from jax import lax
from jax.experimental import pallas as pl
from jax.experimental.pallas import tpu as pltpu
from jax.experimental.pallas import tpu_sc as plsc


@functools.lru_cache(maxsize=None)
def _make_deg(E, N, NC, NS):
  NW = NC * NS
  per = E // NW
  mesh = plsc.VectorSubcoreMesh(core_axis_name="c", subcore_axis_name="s")

  @functools.partial(
      pl.kernel,
      mesh=mesh,
      compiler_params=pltpu.CompilerParams(needs_layout_passes=False),
      out_type=[
          jax.ShapeDtypeStruct((NW, N), jnp.float32),
          jax.ShapeDtypeStruct((E,), jnp.int32),
      ],
      scratch_types=[
          pltpu.VMEM((per,), jnp.int32),
          pltpu.VMEM((per,), jnp.int32),
          pltpu.VMEM((per,), jnp.int32),
          pltpu.VMEM((N,), jnp.float32),
      ],
  )
  def deg_kernel(ei_hbm, out_hbm, pk_hbm, schunk, dchunk, pchunk, cnt):
    wid = lax.axis_index("s") * NC + lax.axis_index("c")
    pltpu.sync_copy(ei_hbm.at[pl.ds(wid * per, per)], schunk)
    pltpu.sync_copy(ei_hbm.at[pl.ds(E + wid * per, per)], dchunk)

    zero = jnp.zeros((16,), jnp.float32)

    @plsc.parallel_loop(0, N // 16, unroll=8)
    def zbody(i):
      cnt[pl.ds(i * 16, 16)] = zero

    ones = jnp.ones((16,), jnp.float32)

    @plsc.parallel_loop(0, per // 16, unroll=8)
    def body(i):
      s = schunk[pl.ds(i * 16, 16)]
      d = dchunk[pl.ds(i * 16, 16)]
      pchunk[pl.ds(i * 16, 16)] = (s << 14) | d
      plsc.addupdate_scatter(cnt, [d], ones)
    pltpu.sync_copy(cnt, out_hbm.at[wid])
    pltpu.sync_copy(pchunk, pk_hbm.at[pl.ds(wid * per, per)])

  return deg_kernel


@functools.lru_cache(maxsize=None)
def _make_scatter(E, N, H, NC, NS, C):
  NW = NC * NS
  R = H // NW
  P = R // 2
  mesh = plsc.VectorSubcoreMesh(core_axis_name="c", subcore_axis_name="s")

  @functools.partial(
      pl.kernel,
      mesh=mesh,
      compiler_params=pltpu.CompilerParams(needs_layout_passes=False),
      out_type=jax.ShapeDtypeStruct((H * N,), jnp.float32),
      scratch_types=[
          pltpu.VMEM((P * N,), jnp.int32),
          pltpu.VMEM((R * N,), jnp.float32),
          pltpu.VMEM((C,), jnp.int32),
          pltpu.VMEM((C,), jnp.int32),
          pltpu.SemaphoreType.DMA,
          pltpu.SemaphoreType.DMA,
      ],
  )
  def scat_kernel(hp_hbm, pk_hbm, out_hbm,
                  hrows, acc, ech0, ech1, sem0, sem1):
    wid = lax.axis_index("s") * NC + lax.axis_index("c")
    pltpu.async_copy(pk_hbm.at[pl.ds(0, C)], ech0, sem0)
    pltpu.sync_copy(hp_hbm.at[pl.ds(wid * (P * N), P * N)], hrows)

    zero = jnp.zeros((16,), jnp.float32)

    @plsc.parallel_loop(0, (R * N) // 16, unroll=8)
    def zbody(i):
      acc[pl.ds(i * 16, 16)] = zero

    nchunk = E // C

    def start(ci, eref, sem):
      pltpu.async_copy(pk_hbm.at[pl.ds(ci * C, C)], eref, sem)

    def waitbuf(eref, sem):
      pltpu.make_async_copy(pk_hbm.at[pl.ds(0, C)], eref, sem).wait()

    himask = jnp.full((16,), -65536, jnp.int32)
    dmask = jnp.full((16,), 16383, jnp.int32)

    def compute(eref):
      @plsc.parallel_loop(0, C // 16, unroll=2)
      def vb(i):
        e = eref[pl.ds(i * 16, 16)]
        s = e >> 14
        d = e & dmask
        for p in range(P):
          v = plsc.load_gather(hrows, [s + (p * N)])
          hi = plsc.bitcast(v & himask, jnp.float32)
          lo = plsc.bitcast(v << 16, jnp.float32)
          plsc.addupdate_scatter(acc, [d + (p * N)], hi)
          plsc.addupdate_scatter(acc, [d + ((P + p) * N)], lo)

    def group(gi, carry):
      c0 = 2 * gi
      start(c0 + 1, ech1, sem1)
      waitbuf(ech0, sem0)
      compute(ech0)
      start(lax.rem(c0 + 2, nchunk), ech0, sem0)
      waitbuf(ech1, sem1)
      compute(ech1)
      return carry

    lax.fori_loop(0, nchunk // 2, group, 0)
    waitbuf(ech0, sem0)
    pltpu.sync_copy(acc.at[pl.ds(0, P * N)],
                    out_hbm.at[pl.ds(wid * (P * N), P * N)])
    pltpu.sync_copy(acc.at[pl.ds(P * N, P * N)],
                    out_hbm.at[pl.ds((NW + wid) * (P * N), P * N)])

  return scat_kernel


def _ka_body(x_ref, w1_ref, b1_ref, cnt_ref, dis_ref, hp_ref):
  z = lax.dot_general(w1_ref[...], x_ref[...], (((1,), (1,)), ((), ())),
                      preferred_element_type=jnp.float32)
  deg = jnp.sum(cnt_ref[...], axis=0, keepdims=True) + 1.0
  dis = lax.rsqrt(deg)
  dis_ref[...] = dis
  hts = jnp.sin(z + b1_ref[...]) * dis
  hh = hts.shape[0] // 2
  top = lax.bitcast_convert_type(
      hts[:hh].astype(jnp.bfloat16), jnp.uint16).astype(jnp.uint32)
  bot = lax.bitcast_convert_type(
      hts[hh:].astype(jnp.bfloat16), jnp.uint16).astype(jnp.uint32)
  hp_ref[...] = lax.bitcast_convert_type((top << 16) | bot, jnp.int32)


def _kb_body(accT_ref, hp_ref, dis_ref, wg_ref, bg_ref, wc_ref, bc_ref, y_ref):
  dis = dis_ref[...]
  hh = hp_ref.shape[0]
  hp = lax.bitcast_convert_type(hp_ref[...], jnp.uint32)
  hi = lax.bitcast_convert_type(hp & jnp.uint32(0xFFFF0000), jnp.float32)
  lo = lax.bitcast_convert_type(hp << 16, jnp.float32)
  m_top = (accT_ref[:hh, :] + hi) * dis
  m_bot = (accT_ref[hh:, :] + lo) * dis
  g = (lax.dot_general(wg_ref[:, :hh], m_top, (((1,), (0,)), ((), ())),
                       preferred_element_type=jnp.float32)
       + lax.dot_general(wg_ref[:, hh:], m_bot, (((1,), (0,)), ((), ())),
                         preferred_element_type=jnp.float32))
  g = jnp.maximum(g + bg_ref[...], 0.0)
  y = lax.dot_general(wc_ref[...], g, (((1,), (0,)), ((), ())),
                      preferred_element_type=jnp.float32)
  y_ref[...] = y + bc_ref[...]


def kernel(x, edge_index, W1, b1, Wg, bg, Wc, bc):
  N, D = x.shape
  H = W1.shape[0]
  OUT = Wc.shape[0]
  E = edge_index.shape[1]
  NC, NS = 2, 16
  NW = NC * NS

  cnt, pk = _make_deg(E, N, NC, NS)(edge_index.reshape(2 * E))

  BN = 512
  grid_a = (pl.cdiv(N, BN),)
  dis2d, hp = pl.pallas_call(
      _ka_body,
      grid=grid_a,
      in_specs=[
          pl.BlockSpec((BN, D), lambda j: (j, 0)),
          pl.BlockSpec((H, D), lambda j: (0, 0)),
          pl.BlockSpec((H, 1), lambda j: (0, 0)),
          pl.BlockSpec((NW, BN), lambda j: (0, j)),
      ],
      out_specs=[
          pl.BlockSpec((1, BN), lambda j: (0, j)),
          pl.BlockSpec((H // 2, BN), lambda j: (0, j)),
      ],
      out_shape=[
          jax.ShapeDtypeStruct((1, N), jnp.float32),
          jax.ShapeDtypeStruct((H // 2, N), jnp.int32),
      ],
  )(x, W1, b1.reshape(H, 1), cnt)

  C = 16000
  accT_flat = _make_scatter(E, N, H, NC, NS, C)(
      hp.reshape((H // 2) * N), pk)
  accT = accT_flat.reshape(H, N)

  grid_b = (pl.cdiv(N, BN),)
  y2d = pl.pallas_call(
      _kb_body,
      grid=grid_b,
      in_specs=[
          pl.BlockSpec((H, BN), lambda j: (0, j)),
          pl.BlockSpec((H // 2, BN), lambda j: (0, j)),
          pl.BlockSpec((1, BN), lambda j: (0, j)),
          pl.BlockSpec((H, H), lambda j: (0, 0)),
          pl.BlockSpec((H, 1), lambda j: (0, 0)),
          pl.BlockSpec((OUT, H), lambda j: (0, 0)),
          pl.BlockSpec((OUT, 1), lambda j: (0, 0)),
      ],
      out_specs=pl.BlockSpec((OUT, BN), lambda j: (0, j)),
      out_shape=jax.ShapeDtypeStruct((OUT, N), jnp.float32),
  )(accT, hp, dis2d, Wg, bg.reshape(H, 1), Wc, bc.reshape(OUT, 1))

  return jnp.squeeze(y2d.T)


# speedup vs baseline: 1.0220x; 1.0026x over previous
import functools

import jax
import jax.numpy as jnp
from jax import lax
from jax.experimental import pallas as pl
from jax.experimental.pallas import tpu as pltpu
from jax.experimental.pallas import tpu_sc as plsc


@functools.lru_cache(maxsize=None)
def _make_deg(E, N, NC, NS):
  NW = NC * NS
  per = E // NW
  mesh = plsc.VectorSubcoreMesh(core_axis_name="c", subcore_axis_name="s")

  @functools.partial(
      pl.kernel,
      mesh=mesh,
      compiler_params=pltpu.CompilerParams(needs_layout_passes=False),
      out_type=[
          jax.ShapeDtypeStruct((NW, N), jnp.float32),
          jax.ShapeDtypeStruct((E,), jnp.int32),
      ],
      scratch_types=[
          pltpu.VMEM((per,), jnp.int32),
          pltpu.VMEM((per,), jnp.int32),
          pltpu.VMEM((per,), jnp.int32),
          pltpu.VMEM((N,), jnp.float32),
      ],
  )
  def deg_kernel(ei_hbm, out_hbm, pk_hbm, schunk, dchunk, pchunk, cnt):
    wid = lax.axis_index("s") * NC + lax.axis_index("c")
    pltpu.sync_copy(ei_hbm.at[pl.ds(wid * per, per)], schunk)
    pltpu.sync_copy(ei_hbm.at[pl.ds(E + wid * per, per)], dchunk)

    zero = jnp.zeros((16,), jnp.float32)

    @plsc.parallel_loop(0, N // 16, unroll=8)
    def zbody(i):
      cnt[pl.ds(i * 16, 16)] = zero

    ones = jnp.ones((16,), jnp.float32)

    @plsc.parallel_loop(0, per // 16, unroll=8)
    def body(i):
      s = schunk[pl.ds(i * 16, 16)]
      d = dchunk[pl.ds(i * 16, 16)]
      pchunk[pl.ds(i * 16, 16)] = (s << 14) | d
      plsc.addupdate_scatter(cnt, [d], ones)
    pltpu.sync_copy(cnt, out_hbm.at[wid])
    pltpu.sync_copy(pchunk, pk_hbm.at[pl.ds(wid * per, per)])

  return deg_kernel


@functools.lru_cache(maxsize=None)
def _make_scatter(E, N, H, NC, NS, C):
  NW = NC * NS
  R = H // NW
  P = R // 2
  mesh = plsc.VectorSubcoreMesh(core_axis_name="c", subcore_axis_name="s")

  @functools.partial(
      pl.kernel,
      mesh=mesh,
      compiler_params=pltpu.CompilerParams(needs_layout_passes=False),
      out_type=jax.ShapeDtypeStruct((H * N,), jnp.float32),
      scratch_types=[
          pltpu.VMEM((P * N,), jnp.int32),
          pltpu.VMEM((R * N,), jnp.float32),
          pltpu.VMEM((C,), jnp.int32),
          pltpu.VMEM((C,), jnp.int32),
          pltpu.SemaphoreType.DMA,
          pltpu.SemaphoreType.DMA,
      ],
  )
  def scat_kernel(hp_hbm, pk_hbm, out_hbm,
                  hrows, acc, ech0, ech1, sem0, sem1):
    wid = lax.axis_index("s") * NC + lax.axis_index("c")
    pltpu.async_copy(pk_hbm.at[pl.ds(0, C)], ech0, sem0)
    pltpu.sync_copy(hp_hbm.at[pl.ds(wid * (P * N), P * N)], hrows)

    zero = jnp.zeros((16,), jnp.float32)

    @plsc.parallel_loop(0, (R * N) // 16, unroll=8)
    def zbody(i):
      acc[pl.ds(i * 16, 16)] = zero

    nchunk = E // C

    def start(ci, eref, sem):
      pltpu.async_copy(pk_hbm.at[pl.ds(ci * C, C)], eref, sem)

    def waitbuf(eref, sem):
      pltpu.make_async_copy(pk_hbm.at[pl.ds(0, C)], eref, sem).wait()

    himask = jnp.full((16,), -65536, jnp.int32)
    dmask = jnp.full((16,), 16383, jnp.int32)

    def compute(eref):
      @plsc.parallel_loop(0, C // 16, unroll=1)
      def vb(i):
        e = eref[pl.ds(i * 16, 16)]
        s = e >> 14
        d = e & dmask
        for p in range(P):
          v = plsc.load_gather(hrows, [s + (p * N)])
          hi = plsc.bitcast(v & himask, jnp.float32)
          lo = plsc.bitcast(v << 16, jnp.float32)
          plsc.addupdate_scatter(acc, [d + (p * N)], hi)
          plsc.addupdate_scatter(acc, [d + ((P + p) * N)], lo)

    def group(gi, carry):
      c0 = 2 * gi
      start(c0 + 1, ech1, sem1)
      waitbuf(ech0, sem0)
      compute(ech0)
      start(lax.rem(c0 + 2, nchunk), ech0, sem0)
      waitbuf(ech1, sem1)
      compute(ech1)
      return carry

    lax.fori_loop(0, nchunk // 2, group, 0)
    waitbuf(ech0, sem0)
    pltpu.sync_copy(acc.at[pl.ds(0, P * N)],
                    out_hbm.at[pl.ds(wid * (P * N), P * N)])
    pltpu.sync_copy(acc.at[pl.ds(P * N, P * N)],
                    out_hbm.at[pl.ds((NW + wid) * (P * N), P * N)])

  return scat_kernel


def _ka_body(x_ref, w1_ref, b1_ref, cnt_ref, dis_ref, hp_ref):
  z = lax.dot_general(w1_ref[...], x_ref[...], (((1,), (1,)), ((), ())),
                      preferred_element_type=jnp.float32)
  deg = jnp.sum(cnt_ref[...], axis=0, keepdims=True) + 1.0
  dis = lax.rsqrt(deg)
  dis_ref[...] = dis
  hts = jnp.sin(z + b1_ref[...]) * dis
  hh = hts.shape[0] // 2
  top = lax.bitcast_convert_type(
      hts[:hh].astype(jnp.bfloat16), jnp.uint16).astype(jnp.uint32)
  bot = lax.bitcast_convert_type(
      hts[hh:].astype(jnp.bfloat16), jnp.uint16).astype(jnp.uint32)
  hp_ref[...] = lax.bitcast_convert_type((top << 16) | bot, jnp.int32)


def _kb_body(accT_ref, hp_ref, dis_ref, wg_ref, bg_ref, wc_ref, bc_ref, y_ref):
  dis = dis_ref[...]
  hh = hp_ref.shape[0]
  hp = lax.bitcast_convert_type(hp_ref[...], jnp.uint32)
  hi = lax.bitcast_convert_type(hp & jnp.uint32(0xFFFF0000), jnp.float32)
  lo = lax.bitcast_convert_type(hp << 16, jnp.float32)
  m_top = (accT_ref[:hh, :] + hi) * dis
  m_bot = (accT_ref[hh:, :] + lo) * dis
  g = (lax.dot_general(wg_ref[:, :hh], m_top, (((1,), (0,)), ((), ())),
                       preferred_element_type=jnp.float32)
       + lax.dot_general(wg_ref[:, hh:], m_bot, (((1,), (0,)), ((), ())),
                         preferred_element_type=jnp.float32))
  g = jnp.maximum(g + bg_ref[...], 0.0)
  y = lax.dot_general(wc_ref[...], g, (((1,), (0,)), ((), ())),
                      preferred_element_type=jnp.float32)
  y_ref[...] = y + bc_ref[...]


def kernel(x, edge_index, W1, b1, Wg, bg, Wc, bc):
  N, D = x.shape
  H = W1.shape[0]
  OUT = Wc.shape[0]
  E = edge_index.shape[1]
  NC, NS = 2, 16
  NW = NC * NS

  cnt, pk = _make_deg(E, N, NC, NS)(edge_index.reshape(2 * E))

  BN = 512
  grid_a = (pl.cdiv(N, BN),)
  dis2d, hp = pl.pallas_call(
      _ka_body,
      grid=grid_a,
      in_specs=[
          pl.BlockSpec((BN, D), lambda j: (j, 0)),
          pl.BlockSpec((H, D), lambda j: (0, 0)),
          pl.BlockSpec((H, 1), lambda j: (0, 0)),
          pl.BlockSpec((NW, BN), lambda j: (0, j)),
      ],
      out_specs=[
          pl.BlockSpec((1, BN), lambda j: (0, j)),
          pl.BlockSpec((H // 2, BN), lambda j: (0, j)),
      ],
      out_shape=[
          jax.ShapeDtypeStruct((1, N), jnp.float32),
          jax.ShapeDtypeStruct((H // 2, N), jnp.int32),
      ],
  )(x, W1, b1.reshape(H, 1), cnt)

  C = 16000
  accT_flat = _make_scatter(E, N, H, NC, NS, C)(
      hp.reshape((H // 2) * N), pk)
  accT = accT_flat.reshape(H, N)

  grid_b = (pl.cdiv(N, BN),)
  y2d = pl.pallas_call(
      _kb_body,
      grid=grid_b,
      in_specs=[
          pl.BlockSpec((H, BN), lambda j: (0, j)),
          pl.BlockSpec((H // 2, BN), lambda j: (0, j)),
          pl.BlockSpec((1, BN), lambda j: (0, j)),
          pl.BlockSpec((H, H), lambda j: (0, 0)),
          pl.BlockSpec((H, 1), lambda j: (0, 0)),
          pl.BlockSpec((OUT, H), lambda j: (0, 0)),
          pl.BlockSpec((OUT, 1), lambda j: (0, 0)),
      ],
      out_specs=pl.BlockSpec((OUT, BN), lambda j: (0, j)),
      out_shape=jax.ShapeDtypeStruct((OUT, N), jnp.float32),
  )(accT, hp, dis2d, Wg, bg.reshape(H, 1), Wc, bc.reshape(OUT, 1))

  return jnp.squeeze(y2d.T)
